# Initial kernel scaffold; baseline (speedup 1.0000x reference)
import jax
import jax.numpy as jnp
from jax.experimental import pallas as pl


def kernel(x, edge_index, batch, embed, eps, W1, b1, W2, b2, hW1, hb1, hW2, hb2):
    raise NotImplementedError("write your pallas kernel here")


# speedup vs baseline: 8.4351x; 8.4351x over previous
import functools

import jax
import jax.numpy as jnp
from jax import lax
from jax.experimental import pallas as pl
from jax.experimental.pallas import tpu as pltpu
from jax.experimental.pallas import tpu_sc as plsc

NN = 50000
EE = 800000
DD = 64
HH = 32
BB = 64
KK = 8
NTT = 51

NC = 2
NS = 16
NP = 51200
CH = 128
UN = 23
NBLK = 17
EPT = NBLK * UN * CH
EP = NS * EPT
ROWS = EP // CH
RPT = NP // NS
TRASH = NN
ZR = 200
RB = 512
GRID = NP // RB

_mesh = plsc.VectorSubcoreMesh(core_axis_name="c", subcore_axis_name="s")


@functools.partial(
    pl.kernel,
    out_type=jax.ShapeDtypeStruct((2 * NP, HH), jnp.float32),
    mesh=_mesh,
    scratch_types=[
        pltpu.VMEM((1, CH), jnp.int32),
        pltpu.VMEM((CH, HH), jnp.float32),
        pltpu.SemaphoreType.DMA,
    ],
    compiler_params=pltpu.CompilerParams(use_tc_tiling_on_sc=False),
)
def _embed_sc(x2_r, emb_r, out_r, xbuf, rows, sem):
    c = lax.axis_index("c")
    s = lax.axis_index("s")
    idx_row0 = c * (NP // CH) + s * (RPT // CH)
    out_base = c * NP + s * RPT

    def body(b, carry):
        pltpu.sync_copy(x2_r.at[pl.ds(idx_row0 + b, 1)], xbuf)
        pltpu.async_copy(emb_r.at[xbuf.at[0]], rows, sem).wait()
        pltpu.sync_copy(rows, out_r.at[pl.ds(out_base + b * CH, CH)])
        return carry

    lax.fori_loop(0, RPT // CH, body, 0)


@functools.partial(
    pl.kernel,
    out_type=jax.ShapeDtypeStruct((2 * NP, HH), jnp.float32),
    mesh=_mesh,
    scratch_types=[
        pltpu.VMEM((UN, CH), jnp.int32),
        pltpu.VMEM((UN, CH), jnp.int32),
        pltpu.VMEM((CH, HH), jnp.float32),
        pltpu.VMEM((CH, HH), jnp.float32),
        pltpu.VMEM((ZR, HH), jnp.float32),
        pltpu.VMEM_SHARED((NP, HH), jnp.float32),
        pltpu.SemaphoreType.DMA,
        pltpu.SemaphoreType.DMA,
    ],
    compiler_params=pltpu.CompilerParams(use_tc_tiling_on_sc=False),
)
def _scatter_sc(h_r, src2_r, dst_r, out_r, sbuf, dbuf, rows0, rows1, zrow,
                acc, sem0, sem1):
    c = lax.axis_index("c")
    s = lax.axis_index("s")

    def zb(i, carry):
        zrow[i, pl.ds(0, 16)] = jnp.zeros((16,), jnp.float32)
        zrow[i, pl.ds(16, 16)] = jnp.zeros((16,), jnp.float32)
        return carry

    lax.fori_loop(0, ZR, zb, 0)

    def za(i, carry):
        pltpu.sync_copy(zrow, acc.at[pl.ds(s * RPT + i * ZR, ZR)])
        return carry

    lax.fori_loop(0, RPT // ZR, za, 0)
    plsc.subcore_barrier()

    bufs = (rows0, rows1)
    sems = (sem0, sem1)

    def blk(b, carry):
        row0 = s * (NBLK * UN) + b * UN
        pltpu.sync_copy(src2_r.at[pl.ds(c * ROWS + row0, UN)], sbuf)
        pltpu.sync_copy(dst_r.at[pl.ds(row0, UN)], dbuf)
        cp = pltpu.async_copy(h_r.at[sbuf.at[0]], bufs[0], sems[0])
        for j in range(UN):
            nxt = None
            if j + 1 < UN:
                p = (j + 1) % 2
                nxt = pltpu.async_copy(h_r.at[sbuf.at[j + 1]], bufs[p], sems[p])
            cp.wait()
            pltpu.sync_copy(bufs[j % 2], acc.at[dbuf.at[j]], add=True)
            cp = nxt
        return carry

    lax.fori_loop(0, NBLK, blk, 0)
    plsc.subcore_barrier()

    pltpu.sync_copy(acc.at[pl.ds(s * RPT, RPT)],
                    out_r.at[pl.ds(c * NP + s * RPT, RPT)])


def _mlp_body(eref, href, aref, w1ref, b1ref, w2ref, b2ref, oref):
    e = eref[0, 0]
    z_a = href[0] * e + aref[0]
    z_b = href[1] * e + aref[1]
    t = jnp.dot(z_a, w1ref[:HH, :], preferred_element_type=jnp.float32)
    t = t + jnp.dot(z_b, w1ref[HH:, :], preferred_element_type=jnp.float32)
    t = jnp.maximum(t + b1ref[...], 0.0)
    o = jnp.dot(t, w2ref[...], preferred_element_type=jnp.float32) + b2ref[...]
    o = jnp.maximum(o, 0.0)
    oref[0] = o[:, :HH]
    oref[1] = o[:, HH:]


def _mlp_call(epsv, h, agg, w1, b1, w2, b2):
    return pl.pallas_call(
        _mlp_body,
        grid=(GRID,),
        in_specs=[
            pl.BlockSpec((1, 1), lambda i: (0, 0)),
            pl.BlockSpec((2, RB, HH), lambda i: (0, i, 0)),
            pl.BlockSpec((2, RB, HH), lambda i: (0, i, 0)),
            pl.BlockSpec((DD, DD), lambda i: (0, 0)),
            pl.BlockSpec((1, DD), lambda i: (0, 0)),
            pl.BlockSpec((DD, DD), lambda i: (0, 0)),
            pl.BlockSpec((1, DD), lambda i: (0, 0)),
        ],
        out_specs=pl.BlockSpec((2, RB, HH), lambda i: (0, i, 0)),
        out_shape=jax.ShapeDtypeStruct((2, NP, HH), jnp.float32),
    )(epsv, h, agg, w1, b1, w2, b2)


def _final_body(eref, href, aref, w1ref, b1ref, w2ref, b2ref, bref,
                hw1ref, hb1ref, hw2ref, hb2ref, oref, sacc):
    i = pl.program_id(0)

    @pl.when(i == 0)
    def _init():
        sacc[...] = jnp.zeros((BB, 2 * DD), jnp.float32)

    e = eref[0, 0]
    z_a = href[0] * e + aref[0]
    z_b = href[1] * e + aref[1]
    t = jnp.dot(z_a, w1ref[:HH, :], preferred_element_type=jnp.float32)
    t = t + jnp.dot(z_b, w1ref[HH:, :], preferred_element_type=jnp.float32)
    t = jnp.maximum(t + b1ref[...], 0.0)
    o = jnp.dot(t, w2ref[...], preferred_element_type=jnp.float32) + b2ref[...]
    o = jnp.maximum(o, 0.0)

    oh = (bref[...] == lax.broadcasted_iota(jnp.int32, (RB, BB), 1))
    oh = oh.astype(jnp.float32)
    ext = jnp.concatenate([o, jnp.ones((RB, DD), jnp.float32)], axis=1)
    sacc[...] += lax.dot_general(oh, ext, (((0,), (0,)), ((), ())),
                                 preferred_element_type=jnp.float32)

    @pl.when(i == GRID - 1)
    def _fin():
        sums = sacc[:, :DD]
        cnt = sacc[:, DD:DD + 1]
        g = sums / jnp.maximum(cnt, 1.0)
        cols = []
        for k in range(KK):
            zh = jnp.maximum(
                jnp.dot(g, hw1ref[k], preferred_element_type=jnp.float32)
                + hb1ref[k], 0.0)
            col = jnp.dot(zh, hw2ref[k], preferred_element_type=jnp.float32)
            cols.append(col + hb2ref[k])
        oref[...] = jnp.concatenate(cols, axis=1)


def _final_call(epsv, h, agg, w1, b1, w2, b2, batch2, hw1, hb1, hw2, hb2):
    return pl.pallas_call(
        _final_body,
        grid=(GRID,),
        in_specs=[
            pl.BlockSpec((1, 1), lambda i: (0, 0)),
            pl.BlockSpec((2, RB, HH), lambda i: (0, i, 0)),
            pl.BlockSpec((2, RB, HH), lambda i: (0, i, 0)),
            pl.BlockSpec((DD, DD), lambda i: (0, 0)),
            pl.BlockSpec((1, DD), lambda i: (0, 0)),
            pl.BlockSpec((DD, DD), lambda i: (0, 0)),
            pl.BlockSpec((1, DD), lambda i: (0, 0)),
            pl.BlockSpec((RB, 1), lambda i: (i, 0)),
            pl.BlockSpec((KK, DD, DD), lambda i: (0, 0, 0)),
            pl.BlockSpec((KK, DD), lambda i: (0, 0)),
            pl.BlockSpec((KK, DD, 1), lambda i: (0, 0, 0)),
            pl.BlockSpec((KK, 1), lambda i: (0, 0)),
        ],
        out_specs=pl.BlockSpec((BB, KK), lambda i: (0, 0)),
        out_shape=jax.ShapeDtypeStruct((BB, KK), jnp.float32),
        scratch_shapes=[pltpu.VMEM((BB, 2 * DD), jnp.float32)],
    )(epsv, h, agg, w1, b1, w2, b2, batch2, hw1, hb1, hw2, hb2)


def kernel(x, edge_index, batch, embed, eps, W1, b1, W2, b2, hW1, hb1, hW2,
           hb2):
    xf = jnp.pad(x[:, 0].astype(jnp.int32), (0, NP - NN))
    x2 = jnp.concatenate([xf, xf + NTT]).reshape(2 * NP // CH, CH)
    embf = jnp.concatenate([embed[:, :HH], embed[:, HH:]], axis=0)

    src = edge_index[0].astype(jnp.int32)
    dst = edge_index[1].astype(jnp.int32)
    srcp = jnp.pad(src, (0, EP - EE))
    dstp = jnp.pad(dst, (0, EP - EE), constant_values=TRASH)
    src2 = jnp.concatenate([srcp, srcp + NP]).reshape(2 * ROWS, CH)
    dst_r = dstp.reshape(ROWS, CH)

    batch2 = jnp.pad(batch.astype(jnp.int32), (0, NP - NN),
                     constant_values=BB).reshape(NP, 1)
    epsv = (1.0 + eps).reshape(-1, 1, 1)

    h = _embed_sc(x2, embf).reshape(2, NP, HH)
    for i in range(2):
        agg = _scatter_sc(h.reshape(2 * NP, HH), src2, dst_r)
        h = _mlp_call(epsv[i], h, agg.reshape(2, NP, HH), W1[i],
                      b1[i].reshape(1, DD), W2[i], b2[i].reshape(1, DD))
    agg = _scatter_sc(h.reshape(2 * NP, HH), src2, dst_r)
    out = _final_call(epsv[2], h, agg.reshape(2, NP, HH), W1[2],
                      b1[2].reshape(1, DD), W2[2], b2[2].reshape(1, DD),
                      batch2, hW1, hb1, hW2.reshape(KK, DD, 1),
                      hb2.reshape(KK, 1))
    return out
